# baseline (speedup 1.0000x reference)
import numpy as np
import jax
import jax.numpy as jnp
from jax import lax
from jax.experimental import pallas as pl
from jax.experimental.pallas import tpu as pltpu
from jax.experimental.pallas import tpu_sc as plsc

_B, _C, _H, _W = 4, 96, 512, 512
_NLOC = 64
_NW = 32
_CL = 128
_VL = 16
_CPW = 16

_N = _B * _C * _H * _W
_N_CHUNKS = _N // _CL
_ROWS_PER_W = _N_CHUNKS // _NW

_idx_tab = np.zeros((_NW, _CPW), np.int32)
_lane_tab = np.full((_NW, _CPW), _CL, np.int32)
_planes_per_w = (_B * _C) // _NW
for _w in range(_NW):
    _idx_tab[_w, :] = (_w + 1) * _ROWS_PER_W - 1
    _slot = 0
    for _p in range(_w * _planes_per_w, (_w + 1) * _planes_per_w):
        _c = _p % _C
        if _c < _NLOC:
            _flat = (_p * _H + (7 * _c) % _H) * _W + (13 * _c) % _W
            _idx_tab[_w, _slot] = _flat // _CL
            _lane_tab[_w, _slot] = _flat % _CL
            _slot += 1
_IDX_TAB = _idx_tab.reshape(_NW * _CPW)
_LANE_TAB = np.broadcast_to(
    _lane_tab.reshape(_NW * _CPW)[:, None], (_NW * _CPW, _VL)).copy()

_sc_mesh = plsc.VectorSubcoreMesh(core_axis_name="c", subcore_axis_name="s")


def _sc_body(x_hbm, idx_hbm, lane_hbm, out_hbm, idx_v, lane_v, chunks_v, sem):
    nc = 2
    wid = lax.axis_index("s") * nc + lax.axis_index("c")
    slab = pl.ds(wid * _ROWS_PER_W, _ROWS_PER_W)
    pltpu.sync_copy(x_hbm.at[slab, :], out_hbm.at[slab, :])

    base = wid * _CPW
    pltpu.sync_copy(idx_hbm.at[pl.ds(base, _CPW)], idx_v)
    pltpu.sync_copy(lane_hbm.at[pl.ds(base, _CPW), :], lane_v)
    pltpu.async_copy(x_hbm.at[idx_v], chunks_v, sem).wait()
    lane = lax.iota(jnp.int32, _VL)
    for j in range(_CPW):
        for k in range(_CL // _VL):
            pos = lane + k * _VL
            sl = pl.ds(k * _VL, _VL)
            chunks_v[j, sl] = jnp.where(
                pos == lane_v[j, :], 0.0, chunks_v[j, sl])
    pltpu.async_copy(chunks_v, out_hbm.at[idx_v], sem).wait()


_sc_watermark = pl.kernel(
    _sc_body,
    out_type=jax.ShapeDtypeStruct((_N_CHUNKS, _CL), jnp.float32),
    mesh=_sc_mesh,
    scratch_types=[
        pltpu.VMEM((_CPW,), jnp.int32),
        pltpu.VMEM((_CPW, _VL), jnp.int32),
        pltpu.VMEM((_CPW, _CL), jnp.float32),
        pltpu.SemaphoreType.DMA,
    ],
    compiler_params=pltpu.CompilerParams(use_tc_tiling_on_sc=False),
)


def kernel(X):
    B, C, H, W = X.shape
    out = _sc_watermark(X.reshape(_N_CHUNKS, _CL),
                        jnp.asarray(_IDX_TAB), jnp.asarray(_LANE_TAB))
    return out.reshape(B, C, H, W)


# speedup vs baseline: 51.4857x; 51.4857x over previous
import jax
import jax.numpy as jnp
from jax.experimental import pallas as pl

_CB = 12


def _body(x_ref, o_ref):
    i = pl.program_id(0)
    x = x_ref[...]
    cb, hh, ww = x.shape
    c_local = jax.lax.broadcasted_iota(jnp.int32, x.shape, 0)
    row = jax.lax.broadcasted_iota(jnp.int32, x.shape, 1)
    col = jax.lax.broadcasted_iota(jnp.int32, x.shape, 2)
    c = (i * cb + c_local) % 96
    cond = (c < 64) & (row == (7 * c) % hh) & (col == (13 * c) % ww)
    o_ref[...] = jnp.where(cond, 0.0, x)


def kernel(X):
    B, C, H, W = X.shape
    Xf = X.reshape(B * C, H, W)
    out = pl.pallas_call(
        _body,
        grid=(B * C // _CB,),
        in_specs=[pl.BlockSpec((_CB, H, W), lambda i: (i, 0, 0))],
        out_specs=pl.BlockSpec((_CB, H, W), lambda i: (i, 0, 0)),
        out_shape=jax.ShapeDtypeStruct((B * C, H, W), X.dtype),
    )(Xf)
    return out.reshape(B, C, H, W)
